# baseline (speedup 1.0000x reference)
import functools

import jax
import jax.numpy as jnp
from jax import lax
from jax.experimental import pallas as pl
from jax.experimental.pallas import tpu as pltpu
from jax.experimental.pallas import tpu_sc as plsc

VOCAB = 1000000
D = 64
B_ROWS = 4096
B_COLS = 200
NW = 32
ROWS_W = B_ROWS // NW
NBUF = 4


DTYPE = jnp.bfloat16


def _embed_body(x_hbm, table_hbm, out_hbm, idx_v, rows, gsems, wsems):
    wid = lax.axis_index("s") * 2 + lax.axis_index("c")
    row0 = wid * ROWS_W
    pltpu.sync_copy(x_hbm.at[pl.ds(row0, ROWS_W)], idx_v)

    def gather(j, b):
        pltpu.async_copy(table_hbm.at[idx_v.at[j]], rows[b], gsems[b])

    for b in range(NBUF):
        gather(b, b)

    def ring(i, _):
        j0 = NBUF * i
        for b in range(NBUF):
            j = j0 + b
            pltpu.make_async_copy(table_hbm.at[idx_v.at[j]], rows[b], gsems[b]).wait()
            pltpu.async_copy(rows[b], out_hbm.at[row0 + j], wsems[b])
            pltpu.make_async_copy(rows[b], out_hbm.at[row0 + j], wsems[b]).wait()
            gather(jnp.minimum(j + NBUF, ROWS_W - 1), b)
        return 0

    lax.fori_loop(0, ROWS_W // NBUF, ring, 0)
    for b in range(NBUF):
        pltpu.make_async_copy(
            table_hbm.at[idx_v.at[ROWS_W - 1]], rows[b], gsems[b]
        ).wait()


@functools.partial(
    pl.kernel,
    out_type=jax.ShapeDtypeStruct((B_ROWS, B_COLS, 2 * D), DTYPE),
    mesh=plsc.VectorSubcoreMesh(core_axis_name="c", subcore_axis_name="s"),
    scratch_types=(
        [pltpu.VMEM((ROWS_W, B_COLS), jnp.int32)]
        + [pltpu.VMEM((B_COLS, 2 * D), DTYPE) for _ in range(NBUF)]
        + [pltpu.SemaphoreType.DMA for _ in range(2 * NBUF)]
    ),
    compiler_params=pltpu.CompilerParams(use_tc_tiling_on_sc=False),
)
def _embed_sc(x_hbm, table_hbm, out_hbm, idx_v, *scratch):
    rows = scratch[:NBUF]
    gsems = scratch[NBUF:2 * NBUF]
    wsems = scratch[2 * NBUF:]
    _embed_body(x_hbm, table_hbm, out_hbm, idx_v, rows, gsems, wsems)


def kernel(x, table):
    tb = table.astype(DTYPE)
    tp = jnp.concatenate([tb, tb], axis=1)
    out = _embed_sc(x.astype(jnp.int32), tp)
    return out[:, :, :D].astype(jnp.float32)


# speedup vs baseline: 2.1769x; 2.1769x over previous
import functools

import jax
import jax.numpy as jnp
from jax import lax
from jax.experimental import pallas as pl
from jax.experimental.pallas import tpu as pltpu
from jax.experimental.pallas import tpu_sc as plsc

VOCAB = 1000000
D = 64
B_ROWS = 4096
B_COLS = 200
NW = 32
ROWS_W = B_ROWS // NW
NBUF = 8


def _embed_body(x_hbm, table_hbm, out_hbm, idx_v, rows, gsems, wsems):
    wid = lax.axis_index("s") * 2 + lax.axis_index("c")
    row0 = wid * ROWS_W
    pltpu.sync_copy(x_hbm.at[pl.ds(row0, ROWS_W)], idx_v)

    def gather(j, b):
        pltpu.async_copy(table_hbm.at[idx_v.at[j]], rows[b], gsems[b])

    for b in range(NBUF):
        gather(b, b)

    def ring(i, _):
        j0 = NBUF * i
        for b in range(NBUF):
            j = j0 + b
            pltpu.make_async_copy(table_hbm.at[idx_v.at[j]], rows[b], gsems[b]).wait()
            pltpu.async_copy(rows[b], out_hbm.at[row0 + j], wsems[b])
            pltpu.make_async_copy(rows[b], out_hbm.at[row0 + j], wsems[b]).wait()
            gather(jnp.minimum(j + NBUF, ROWS_W - 1), b)
        return 0

    lax.fori_loop(0, ROWS_W // NBUF, ring, 0)
    for b in range(NBUF):
        pltpu.make_async_copy(
            table_hbm.at[idx_v.at[ROWS_W - 1]], rows[b], gsems[b]
        ).wait()


@functools.partial(
    pl.kernel,
    out_type=jax.ShapeDtypeStruct((B_ROWS, B_COLS, D), jnp.float32),
    mesh=plsc.VectorSubcoreMesh(core_axis_name="c", subcore_axis_name="s"),
    scratch_types=(
        [pltpu.VMEM((ROWS_W, B_COLS), jnp.int32)]
        + [pltpu.VMEM((B_COLS, D), jnp.float32) for _ in range(NBUF)]
        + [pltpu.SemaphoreType.DMA for _ in range(2 * NBUF)]
    ),
    compiler_params=pltpu.CompilerParams(use_tc_tiling_on_sc=False),
)
def _embed_sc(x_hbm, table_hbm, out_hbm, idx_v, *scratch):
    rows = scratch[:NBUF]
    gsems = scratch[NBUF:2 * NBUF]
    wsems = scratch[2 * NBUF:]
    _embed_body(x_hbm, table_hbm, out_hbm, idx_v, rows, gsems, wsems)


def kernel(x, table):
    return _embed_sc(x.astype(jnp.int32), table)


# speedup vs baseline: 2.1818x; 1.0023x over previous
import functools

import jax
import jax.numpy as jnp
from jax import lax
from jax.experimental import pallas as pl
from jax.experimental.pallas import tpu as pltpu
from jax.experimental.pallas import tpu_sc as plsc

VOCAB = 1000000
D = 64
B_ROWS = 4096
B_COLS = 200
NW = 32
ROWS_W = B_ROWS // NW
NBUF = 4


def _embed_body(x_hbm, table_hbm, out_hbm, idx_v, rows, gsems, wsems):
    wid = lax.axis_index("s") * 2 + lax.axis_index("c")
    row0 = wid * ROWS_W
    pltpu.sync_copy(x_hbm.at[pl.ds(row0, ROWS_W)], idx_v)

    def gather(j, b):
        pltpu.async_copy(table_hbm.at[idx_v.at[j]], rows[b], gsems[b])

    for b in range(NBUF):
        gather(b, b)

    def ring(i, _):
        j0 = NBUF * i
        for b in range(NBUF):
            j = j0 + b
            pltpu.make_async_copy(table_hbm.at[idx_v.at[j]], rows[b], gsems[b]).wait()
            pltpu.async_copy(rows[b], out_hbm.at[row0 + j], wsems[b])
            pltpu.make_async_copy(rows[b], out_hbm.at[row0 + j], wsems[b]).wait()
            gather(jnp.minimum(j + NBUF, ROWS_W - 1), b)
        return 0

    lax.fori_loop(0, ROWS_W // NBUF, ring, 0)
    for b in range(NBUF):
        pltpu.make_async_copy(
            table_hbm.at[idx_v.at[ROWS_W - 1]], rows[b], gsems[b]
        ).wait()


@functools.partial(
    pl.kernel,
    out_type=jax.ShapeDtypeStruct((B_ROWS, B_COLS, D), jnp.float32),
    mesh=plsc.VectorSubcoreMesh(core_axis_name="c", subcore_axis_name="s"),
    scratch_types=(
        [pltpu.VMEM((ROWS_W, B_COLS), jnp.int32)]
        + [pltpu.VMEM((B_COLS, D), jnp.float32) for _ in range(NBUF)]
        + [pltpu.SemaphoreType.DMA for _ in range(2 * NBUF)]
    ),
    compiler_params=pltpu.CompilerParams(use_tc_tiling_on_sc=False),
)
def _embed_sc(x_hbm, table_hbm, out_hbm, idx_v, *scratch):
    rows = scratch[:NBUF]
    gsems = scratch[NBUF:2 * NBUF]
    wsems = scratch[2 * NBUF:]
    _embed_body(x_hbm, table_hbm, out_hbm, idx_v, rows, gsems, wsems)


def kernel(x, table):
    return _embed_sc(x.astype(jnp.int32), table)
